# baseline (speedup 1.0000x reference)
import functools

import jax
import jax.numpy as jnp
from jax import lax
from jax.experimental import pallas as pl
from jax.experimental.pallas import tpu as pltpu
from jax.experimental.pallas import tpu_sc as plsc

_MARGIN = 0.1
_N = 1_000_000

_LANES = 16
_NCOL = 32
_RCOL = 31_488
_NPAD = _NCOL * _RCOL
_SEG = _RCOL // _NCOL
_VPC = _RCOL // _LANES
_HALF = _RCOL // 2
_H2V = _HALF // _LANES

_PASSES = ((0, 0x7FF), (11, 0x7FF), (22, 0x3FF))
_R = 2048

_G = 7872
_C = 128


def _desc_key(bits):
  neg = (bits >> jnp.uint32(31)) == jnp.uint32(1)
  return jnp.where(neg, bits, (~bits) & jnp.uint32(0x7FFFFFFF))


def _undesc_key(key):
  neg = (key >> jnp.uint32(31)) == jnp.uint32(1)
  return jnp.where(neg, key, (~key) & jnp.uint32(0x7FFFFFFF))


def _sc_sort_body(yp_hbm, yt_hbm, sp_hbm, st_hbm, stage, tmpb,
                  x, y, histcnt, histb):
  c = lax.axis_index("c")
  s = lax.axis_index("s")
  iota = lax.iota(jnp.int32, _LANES)

  def zero_hist(h):
    @pl.loop(0, _R // _LANES, unroll=8)
    def _(i):
      h[pl.ds(i * _LANES, _LANES)] = jnp.zeros((_LANES,), jnp.int32)

  def local_sort(convert_first):
    for p, (shift, mask) in enumerate(_PASSES):
      src, dst = (x, y) if p % 2 == 0 else (y, x)
      convert = convert_first and p == 0
      zero_hist(histcnt)
      zero_hist(histb)

      @pl.loop(0, _H2V, unroll=2)
      def _(v):
        for off, h in ((0, histcnt), (_HALF, histb)):
          kk = lax.bitcast_convert_type(
              src[pl.ds(off + v * _LANES, _LANES)], jnp.uint32)
          if convert:
            kk = _desc_key(kk)
          d = ((kk >> jnp.uint32(shift)) & jnp.uint32(mask)).astype(jnp.int32)
          occ, lastm = plsc.scan_count(d)
          plsc.addupdate_scatter(h, [d], occ + 1, mask=lastm)

      @pl.loop(0, _R // _LANES, init_carry=jnp.int32(0), unroll=4)
      def _(i, carry):
        sl = pl.ds(i * _LANES, _LANES)
        t0 = histcnt[sl]
        t1 = histb[sl]
        tot = t0 + t1
        e = carry + plsc.cumsum(tot) - tot
        histcnt[sl] = e
        histb[sl] = e + t0
        return carry + jnp.sum(tot)

      @pl.loop(0, _H2V, unroll=2)
      def _(v):
        for off, h in ((0, histcnt), (_HALF, histb)):
          kk = lax.bitcast_convert_type(
              src[pl.ds(off + v * _LANES, _LANES)], jnp.uint32)
          if convert:
            kk = _desc_key(kk)
          d = ((kk >> jnp.uint32(shift)) & jnp.uint32(mask)).astype(jnp.int32)
          occ, lastm = plsc.scan_count(d)
          cnt = plsc.load_gather(h, [d])
          pos = cnt + occ
          plsc.store_scatter(
              dst, [jnp.minimum(jnp.maximum(pos, 0), _RCOL - 1)],
              lax.bitcast_convert_type(kk, jnp.float32))
          plsc.store_scatter(h, [d], pos + 1, mask=lastm)

  def load_input_col(k):
    @pl.when(c == 0)
    def _():
      pltpu.sync_copy(yp_hbm.at[pl.ds(k * _RCOL, _RCOL)], x)

    @pl.when(c != 0)
    def _():
      pltpu.sync_copy(yt_hbm.at[pl.ds(k * _RCOL, _RCOL)], x)

  for k in (2 * s, 2 * s + 1):
    load_input_col(k)
    local_sort(convert_first=True)

    @pl.loop(0, _VPC)
    def _(v):
      vals = y[pl.ds(v * _LANES, _LANES)]
      idx = ((v % 2) * _LANES + iota) * _SEG + v // 2
      plsc.store_scatter(x, [idx], vals)

    @pl.loop(0, _NCOL)
    def _(t):
      pltpu.sync_copy(x.at[pl.ds(t * _SEG, _SEG)],
                      stage.at[pl.ds(t * _RCOL + k * _SEG, _SEG)])

  plsc.subcore_barrier()

  for k in (2 * s, 2 * s + 1):
    pltpu.sync_copy(stage.at[pl.ds(k * _RCOL, _RCOL)], x)
    local_sort(convert_first=False)

    @pl.loop(0, _NCOL)
    def _(t):
      pltpu.sync_copy(y.at[pl.ds(t * _SEG, _SEG)],
                      tmpb.at[pl.ds(t * _RCOL + k * _SEG, _SEG)])

  plsc.subcore_barrier()

  def out_slice(k, off=0):
    return pl.ds(k * _RCOL + off, _RCOL)

  for k in (2 * s, 2 * s + 1):
    pltpu.sync_copy(tmpb.at[pl.ds(k * _RCOL, _RCOL)], x)
    local_sort(convert_first=False)

    @pl.when(c == 0)
    def _():
      pltpu.sync_copy(y, sp_hbm.at[out_slice(k)])

    @pl.when(c != 0)
    def _():
      pltpu.sync_copy(y, st_hbm.at[out_slice(k)])

  plsc.subcore_barrier()

  for k in (2 * s, 2 * s + 1):
    @pl.when(k < _NCOL - 1)
    def _():
      @pl.when(c == 0)
      def _():
        pltpu.sync_copy(sp_hbm.at[out_slice(k, _HALF)], x)

      @pl.when(c != 0)
      def _():
        pltpu.sync_copy(st_hbm.at[out_slice(k, _HALF)], x)

      local_sort(convert_first=False)

      @pl.when(c == 0)
      def _():
        pltpu.sync_copy(y, sp_hbm.at[out_slice(k, _HALF)])

      @pl.when(c != 0)
      def _():
        pltpu.sync_copy(y, st_hbm.at[out_slice(k, _HALF)])


_sc_sort = pl.kernel(
    _sc_sort_body,
    out_type=(
        jax.ShapeDtypeStruct((_NPAD,), jnp.float32),
        jax.ShapeDtypeStruct((_NPAD,), jnp.float32),
    ),
    mesh=plsc.VectorSubcoreMesh(
        core_axis_name="c", subcore_axis_name="s", num_cores=2, num_subcores=16
    ),
    compiler_params=pltpu.CompilerParams(needs_layout_passes=False),
    scratch_types=[
        pltpu.VMEM_SHARED((_NPAD,), jnp.float32),
        pltpu.HBM((_NPAD,), jnp.float32),
        pltpu.VMEM((_RCOL,), jnp.float32),
        pltpu.VMEM((_RCOL,), jnp.float32),
        pltpu.VMEM((_R,), jnp.int32),
        pltpu.VMEM((_R,), jnp.int32),
    ],
)


def _tc_mean_body(t_ref, p_ref, o_ref):
  t = lax.bitcast_convert_type(
      _undesc_key(lax.bitcast_convert_type(t_ref[...], jnp.uint32)),
      jnp.float32)
  p = lax.bitcast_convert_type(
      _undesc_key(lax.bitcast_convert_type(p_ref[...], jnp.uint32)),
      jnp.float32)
  tn = jnp.concatenate(
      [t[:, 1:], jnp.concatenate([t[1:, :1], t[:1, :1]], axis=0)], axis=1)
  pn = jnp.concatenate(
      [p[:, 1:], jnp.concatenate([p[1:, :1], p[:1, :1]], axis=0)], axis=1)
  term = jnp.maximum(_MARGIN - (t - tn) + (p - pn), 0.0)
  idx = (lax.broadcasted_iota(jnp.int32, (_G, _C), 0) * _C
         + lax.broadcasted_iota(jnp.int32, (_G, _C), 1))
  term = jnp.where(idx < _N - 1, term, 0.0)
  o_ref[...] = (jnp.sum(term) / (_N - 1)).reshape(1, 1)


_tc_mean = pl.pallas_call(
    _tc_mean_body,
    out_shape=jax.ShapeDtypeStruct((1, 1), jnp.float32),
)


def kernel(y_pred, y_true):
  pad = jnp.full((_NPAD - _N,), -jnp.inf, dtype=jnp.float32)
  yp = jnp.concatenate([y_pred.astype(jnp.float32), pad])
  yt = jnp.concatenate([y_true.astype(jnp.float32), pad])
  sp, st = _sc_sort(yp, yt)
  out = _tc_mean(st.reshape(_G, _C), sp.reshape(_G, _C))
  return out[0, 0]


# speedup vs baseline: 1.3991x; 1.3991x over previous
import functools

import jax
import jax.numpy as jnp
from jax import lax
from jax.experimental import pallas as pl
from jax.experimental.pallas import tpu as pltpu
from jax.experimental.pallas import tpu_sc as plsc

_MARGIN = 0.1
_N = 1_000_000

_LANES = 16
_NCOL = 32
_RCOL = 31_488
_NPAD = _NCOL * _RCOL
_SEG = _RCOL // _NCOL
_VPC = _RCOL // _LANES
_HALF = _RCOL // 2
_H2V = _HALF // _LANES

_PASSES = ((0, 0x7FF), (11, 0x7FF), (22, 0x3FF))
_R = 2048

_G = 7872
_C = 128


def _desc_key(bits):
  neg = (bits >> jnp.uint32(31)) == jnp.uint32(1)
  return jnp.where(neg, bits, (~bits) & jnp.uint32(0x7FFFFFFF))


def _undesc_key(key):
  neg = (key >> jnp.uint32(31)) == jnp.uint32(1)
  return jnp.where(neg, key, (~key) & jnp.uint32(0x7FFFFFFF))


def _sc_sort_body(yp_hbm, yt_hbm, sp_hbm, st_hbm, stage, tmpb,
                  x, y, histcnt, histb):
  c = lax.axis_index("c")
  s = lax.axis_index("s")
  iota = lax.iota(jnp.int32, _LANES)

  def zero_hist(h):
    @pl.loop(0, _R // _LANES, unroll=8)
    def _(i):
      h[pl.ds(i * _LANES, _LANES)] = jnp.zeros((_LANES,), jnp.int32)

  def local_sort(convert_first):
    hists = (histcnt, histb)
    zero_hist(histcnt)
    shift0, mask0 = _PASSES[0]

    @pl.loop(0, _VPC, unroll=4)
    def _(v):
      kk = lax.bitcast_convert_type(x[pl.ds(v * _LANES, _LANES)], jnp.uint32)
      if convert_first:
        kk = _desc_key(kk)
      d = ((kk >> jnp.uint32(shift0)) & jnp.uint32(mask0)).astype(jnp.int32)
      occ, lastm = plsc.scan_count(d)
      plsc.addupdate_scatter(histcnt, [d], occ + 1, mask=lastm)

    for p, (shift, mask) in enumerate(_PASSES):
      src, dst = (x, y) if p % 2 == 0 else (y, x)
      convert = convert_first and p == 0
      ha = hists[p % 2]
      hb = hists[(p + 1) % 2]
      last_pass = p == len(_PASSES) - 1

      @pl.loop(0, _R // _LANES, init_carry=jnp.int32(0), unroll=4)
      def _(i, carry):
        sl = pl.ds(i * _LANES, _LANES)
        t = ha[sl]
        ha[sl] = carry + plsc.cumsum(t) - t
        return carry + jnp.sum(t)

      if not last_pass:
        zero_hist(hb)

      @pl.loop(0, _VPC, unroll=4)
      def _(v):
        kk = lax.bitcast_convert_type(src[pl.ds(v * _LANES, _LANES)],
                                      jnp.uint32)
        if convert:
          kk = _desc_key(kk)
        d = ((kk >> jnp.uint32(shift)) & jnp.uint32(mask)).astype(jnp.int32)
        occ, lastm = plsc.scan_count(d)
        cnt = plsc.load_gather(ha, [d])
        pos = cnt + occ
        plsc.store_scatter(
            dst, [jnp.minimum(jnp.maximum(pos, 0), _RCOL - 1)],
            lax.bitcast_convert_type(kk, jnp.float32))
        plsc.store_scatter(ha, [d], pos + 1, mask=lastm)
        if not last_pass:
          s2, m2 = _PASSES[p + 1]
          d2 = ((kk >> jnp.uint32(s2)) & jnp.uint32(m2)).astype(jnp.int32)
          occ2, lastm2 = plsc.scan_count(d2)
          plsc.addupdate_scatter(hb, [d2], occ2 + 1, mask=lastm2)

  def load_input_col(k):
    @pl.when(c == 0)
    def _():
      pltpu.sync_copy(yp_hbm.at[pl.ds(k * _RCOL, _RCOL)], x)

    @pl.when(c != 0)
    def _():
      pltpu.sync_copy(yt_hbm.at[pl.ds(k * _RCOL, _RCOL)], x)

  for k in (2 * s, 2 * s + 1):
    load_input_col(k)
    local_sort(convert_first=True)

    @pl.loop(0, _VPC)
    def _(v):
      vals = y[pl.ds(v * _LANES, _LANES)]
      idx = ((v % 2) * _LANES + iota) * _SEG + v // 2
      plsc.store_scatter(x, [idx], vals)

    @pl.loop(0, _NCOL)
    def _(t):
      pltpu.sync_copy(x.at[pl.ds(t * _SEG, _SEG)],
                      stage.at[pl.ds(t * _RCOL + k * _SEG, _SEG)])

  plsc.subcore_barrier()

  for k in (2 * s, 2 * s + 1):
    pltpu.sync_copy(stage.at[pl.ds(k * _RCOL, _RCOL)], x)
    local_sort(convert_first=False)

    @pl.loop(0, _NCOL)
    def _(t):
      pltpu.sync_copy(y.at[pl.ds(t * _SEG, _SEG)],
                      tmpb.at[pl.ds(t * _RCOL + k * _SEG, _SEG)])

  plsc.subcore_barrier()

  def out_slice(k, off=0):
    return pl.ds(k * _RCOL + off, _RCOL)

  for k in (2 * s, 2 * s + 1):
    pltpu.sync_copy(tmpb.at[pl.ds(k * _RCOL, _RCOL)], x)
    local_sort(convert_first=False)

    @pl.when(c == 0)
    def _():
      pltpu.sync_copy(y, sp_hbm.at[out_slice(k)])

    @pl.when(c != 0)
    def _():
      pltpu.sync_copy(y, st_hbm.at[out_slice(k)])

  plsc.subcore_barrier()

  for k in (2 * s, 2 * s + 1):
    @pl.when(k < _NCOL - 1)
    def _():
      @pl.when(c == 0)
      def _():
        pltpu.sync_copy(sp_hbm.at[out_slice(k, _HALF)], x)

      @pl.when(c != 0)
      def _():
        pltpu.sync_copy(st_hbm.at[out_slice(k, _HALF)], x)

      local_sort(convert_first=False)

      @pl.when(c == 0)
      def _():
        pltpu.sync_copy(y, sp_hbm.at[out_slice(k, _HALF)])

      @pl.when(c != 0)
      def _():
        pltpu.sync_copy(y, st_hbm.at[out_slice(k, _HALF)])


_sc_sort = pl.kernel(
    _sc_sort_body,
    out_type=(
        jax.ShapeDtypeStruct((_NPAD,), jnp.float32),
        jax.ShapeDtypeStruct((_NPAD,), jnp.float32),
    ),
    mesh=plsc.VectorSubcoreMesh(
        core_axis_name="c", subcore_axis_name="s", num_cores=2, num_subcores=16
    ),
    compiler_params=pltpu.CompilerParams(needs_layout_passes=False),
    scratch_types=[
        pltpu.VMEM_SHARED((_NPAD,), jnp.float32),
        pltpu.HBM((_NPAD,), jnp.float32),
        pltpu.VMEM((_RCOL,), jnp.float32),
        pltpu.VMEM((_RCOL,), jnp.float32),
        pltpu.VMEM((_R,), jnp.int32),
        pltpu.VMEM((_R,), jnp.int32),
    ],
)


def _tc_mean_body(t_ref, p_ref, o_ref):
  t = lax.bitcast_convert_type(
      _undesc_key(lax.bitcast_convert_type(t_ref[...], jnp.uint32)),
      jnp.float32)
  p = lax.bitcast_convert_type(
      _undesc_key(lax.bitcast_convert_type(p_ref[...], jnp.uint32)),
      jnp.float32)
  tn = jnp.concatenate(
      [t[:, 1:], jnp.concatenate([t[1:, :1], t[:1, :1]], axis=0)], axis=1)
  pn = jnp.concatenate(
      [p[:, 1:], jnp.concatenate([p[1:, :1], p[:1, :1]], axis=0)], axis=1)
  term = jnp.maximum(_MARGIN - (t - tn) + (p - pn), 0.0)
  idx = (lax.broadcasted_iota(jnp.int32, (_G, _C), 0) * _C
         + lax.broadcasted_iota(jnp.int32, (_G, _C), 1))
  term = jnp.where(idx < _N - 1, term, 0.0)
  o_ref[...] = (jnp.sum(term) / (_N - 1)).reshape(1, 1)


_tc_mean = pl.pallas_call(
    _tc_mean_body,
    out_shape=jax.ShapeDtypeStruct((1, 1), jnp.float32),
)


def kernel(y_pred, y_true):
  pad = jnp.full((_NPAD - _N,), -jnp.inf, dtype=jnp.float32)
  yp = jnp.concatenate([y_pred.astype(jnp.float32), pad])
  yt = jnp.concatenate([y_true.astype(jnp.float32), pad])
  sp, st = _sc_sort(yp, yt)
  out = _tc_mean(st.reshape(_G, _C), sp.reshape(_G, _C))
  return out[0, 0]
